# baseline (speedup 1.0000x reference)
import functools

import jax
import jax.numpy as jnp
from jax import lax
from jax.experimental import pallas as pl
from jax.experimental.pallas import tpu as pltpu
from jax.experimental.pallas import tpu_sc as plsc

TOPK = 2
TM = 256


def _router_body(x_ref, gw_ref, idx_ref, w_ref):
    x = x_ref[...]
    gw = gw_ref[...]
    logits = lax.dot_general(x, gw, (((1,), (1,)), ((), ())),
                             preferred_element_type=jnp.float32)
    T, E = logits.shape
    col = lax.broadcasted_iota(jnp.int32, (T, E), 1)
    m1 = jnp.max(logits, axis=1, keepdims=True)
    i1 = jnp.min(jnp.where(logits == m1, col, E), axis=1, keepdims=True)
    l2 = jnp.where(col == i1, -jnp.inf, logits)
    m2 = jnp.max(l2, axis=1, keepdims=True)
    i2 = jnp.min(jnp.where(l2 == m2, col, E), axis=1, keepdims=True)
    p1 = 1.0 / (1.0 + jnp.exp(m2 - m1))
    idx_ref[...] = jnp.concatenate([i1, i2], axis=1)
    w_ref[...] = jnp.concatenate([p1, 1.0 - p1], axis=1)


def _router(x, gate_w):
    T = x.shape[0]
    return pl.pallas_call(
        _router_body,
        out_shape=(jax.ShapeDtypeStruct((T, TOPK), jnp.int32),
                   jax.ShapeDtypeStruct((T, TOPK), jnp.float32)),
    )(x, gate_w)


def _sc_dispatch(x, tok, pos, wsrc, P):
    A = tok.shape[0]
    D = x.shape[1]
    L = wsrc.shape[1]
    info = plsc.get_sparse_core_info()
    NC, NS = info.num_cores, info.num_subcores
    NW = NC * NS
    per_w = A // NW
    CH = 32
    n_chunks = per_w // CH
    pos3 = pos.reshape(NW, n_chunks, CH)
    w4 = wsrc.reshape(NW, n_chunks, CH, L)
    mesh = plsc.VectorSubcoreMesh(core_axis_name="c", subcore_axis_name="s")

    @functools.partial(
        pl.kernel, mesh=mesh,
        out_type=(jax.ShapeDtypeStruct((P, D), jnp.float32),
                  jax.ShapeDtypeStruct((P, L), jnp.float32)),
        scratch_types=[pltpu.VMEM((per_w,), jnp.int32),
                       pltpu.VMEM((CH,), jnp.int32),
                       pltpu.VMEM((CH,), jnp.int32),
                       pltpu.VMEM((CH, L), jnp.float32),
                       pltpu.VMEM((2 * CH, D), jnp.float32),
                       pltpu.SemaphoreType.DMA,
                       pltpu.SemaphoreType.DMA,
                       pltpu.SemaphoreType.DMA,
                       pltpu.SemaphoreType.DMA],
    )
    def dispatch_k(x_hbm, tok_hbm, pos_hbm, w_hbm, out_hbm, wr_hbm,
                   tok_v, pos_v0, pos_v1, w_v, rows_v,
                   gsem0, gsem1, ssem0, ssem1):
        wid = lax.axis_index("s") * NC + lax.axis_index("c")
        base = wid * per_w
        pltpu.sync_copy(tok_hbm.at[pl.ds(base, per_w)], tok_v)
        gsems = (gsem0, gsem1)
        ssems = (ssem0, ssem1)
        posv = (pos_v0, pos_v1)

        def gdesc(i):
            buf = rows_v.at[pl.ds((i % 2) * CH, CH)]
            return pltpu.make_async_copy(
                x_hbm.at[tok_v.at[pl.ds(i * CH, CH)]], buf, gsems[i % 2])

        def sdesc(i):
            buf = rows_v.at[pl.ds((i % 2) * CH, CH)]
            return pltpu.make_async_copy(
                buf, out_hbm.at[posv[i % 2]], ssems[i % 2])

        gdesc(0).start()
        for i in range(n_chunks):
            if i + 1 < n_chunks:
                if i >= 1:
                    sdesc(i - 1).wait()
                gdesc(i + 1).start()
            gdesc(i).wait()
            pltpu.sync_copy(pos_hbm.at[wid, i], posv[i % 2])
            sdesc(i).start()
            pltpu.sync_copy(w_hbm.at[wid, i], w_v)
            pltpu.sync_copy(w_v, wr_hbm.at[posv[i % 2]])
        if n_chunks >= 2:
            sdesc(n_chunks - 2).wait()
        sdesc(n_chunks - 1).wait()

    return dispatch_k(x, tok, pos3, w4)


def _mlp_body(tile_e_ref, valid_ref, xs_ref, w1a_ref, w1b_ref, w3a_ref,
              w3b_ref, w2a_ref, w2b_ref, wr_ref, ys_ref):
    t = pl.program_id(0)

    @pl.when(valid_ref[t] > 0)
    def _():
        xs = xs_ref[...]
        cdim = (((1,), (1,)), ((), ()))
        a1 = lax.dot_general(xs, w1a_ref[0], cdim,
                             preferred_element_type=jnp.float32)
        a2 = lax.dot_general(xs, w1b_ref[0], cdim,
                             preferred_element_type=jnp.float32)
        b1 = lax.dot_general(xs, w3a_ref[0], cdim,
                             preferred_element_type=jnp.float32)
        b2 = lax.dot_general(xs, w3b_ref[0], cdim,
                             preferred_element_type=jnp.float32)
        h1 = (a1 * jax.nn.sigmoid(a1)) * b1
        h2 = (a2 * jax.nn.sigmoid(a2)) * b2
        y = (lax.dot_general(h1, w2a_ref[0], cdim,
                             preferred_element_type=jnp.float32)
             + lax.dot_general(h2, w2b_ref[0], cdim,
                               preferred_element_type=jnp.float32))
        ys_ref[...] = y * wr_ref[:, 0:1]


def _grouped_mlp(xs, w1, w3, w2, wr, tile_e, tile_valid, NT):
    P, d = xs.shape
    E, dff, _ = w1.shape
    dff2 = dff // 2
    grid_spec = pltpu.PrefetchScalarGridSpec(
        num_scalar_prefetch=2,
        grid=(NT,),
        in_specs=[
            pl.BlockSpec((TM, d), lambda i, te, va: (i, 0)),
            pl.BlockSpec((1, dff2, d), lambda i, te, va: (te[i], 0, 0)),
            pl.BlockSpec((1, dff2, d), lambda i, te, va: (te[i], 1, 0)),
            pl.BlockSpec((1, dff2, d), lambda i, te, va: (te[i], 0, 0)),
            pl.BlockSpec((1, dff2, d), lambda i, te, va: (te[i], 1, 0)),
            pl.BlockSpec((1, d, dff2), lambda i, te, va: (te[i], 0, 0)),
            pl.BlockSpec((1, d, dff2), lambda i, te, va: (te[i], 0, 1)),
            pl.BlockSpec((TM, 128), lambda i, te, va: (i, 0)),
        ],
        out_specs=pl.BlockSpec((TM, d), lambda i, te, va: (i, 0)),
    )
    return pl.pallas_call(
        _mlp_body,
        grid_spec=grid_spec,
        out_shape=jax.ShapeDtypeStruct((P, d), jnp.float32),
    )(tile_e, tile_valid, xs, w1, w1, w3, w3, w2, w2, wr)


def _sc_combine(ys, pos0, pos1):
    T = pos0.shape[0]
    D = ys.shape[1]
    info = plsc.get_sparse_core_info()
    NC, NS = info.num_cores, info.num_subcores
    NW = NC * NS
    per_w = T // NW
    CH = 32 if per_w % 32 == 0 else 16
    n_chunks = per_w // CH
    mesh = plsc.VectorSubcoreMesh(core_axis_name="c", subcore_axis_name="s")

    @functools.partial(
        pl.kernel, mesh=mesh,
        out_type=jax.ShapeDtypeStruct((T, D), jnp.float32),
        scratch_types=[pltpu.VMEM((CH,), jnp.int32),
                       pltpu.VMEM((CH,), jnp.int32),
                       pltpu.VMEM((CH, D), jnp.float32),
                       pltpu.VMEM((CH, D), jnp.float32),
                       pltpu.SemaphoreType.DMA],
    )
    def combine_k(ys_hbm, p0_hbm, p1_hbm, out_hbm,
                  p0_v, p1_v, r0_v, r1_v, sem):
        wid = lax.axis_index("s") * NC + lax.axis_index("c")
        base = wid * per_w

        def chunk(c, carry):
            off = base + c * CH
            pltpu.sync_copy(p0_hbm.at[pl.ds(off, CH)], p0_v)
            pltpu.sync_copy(p1_hbm.at[pl.ds(off, CH)], p1_v)
            pltpu.async_copy(ys_hbm.at[p0_v], r0_v, sem).wait()
            pltpu.async_copy(ys_hbm.at[p1_v], r1_v, sem).wait()

            def row(i, rcarry):
                for j in range(D // 16):
                    sl = pl.ds(j * 16, 16)
                    r0_v[i, sl] = r0_v[i, sl] + r1_v[i, sl]
                return rcarry

            lax.fori_loop(0, CH, row, 0)
            pltpu.sync_copy(r0_v, out_hbm.at[pl.ds(off, CH)])
            return carry

        lax.fori_loop(0, n_chunks, chunk, 0)

    return combine_k(ys, pos0, pos1)


def kernel(hidden_states, gate_w, w1, w3, w2):
    orig_shape = hidden_states.shape
    d = orig_shape[-1]
    x = hidden_states.reshape(-1, d)
    T = x.shape[0]
    E = gate_w.shape[0]
    A = T * TOPK
    NT = A // TM + E
    P = NT * TM

    top_idx, top_w = _router(x, gate_w)

    i32 = jnp.int32
    f32 = jnp.float32
    e_a = top_idx.T.reshape(-1)
    onehot = (e_a[:, None] == jnp.arange(E, dtype=i32)[None, :]).astype(f32)
    cum = jnp.cumsum(onehot, axis=0)
    rank_a = jnp.sum((cum - onehot) * onehot, axis=1)
    counts = cum[A - 1]
    tiles_per = jnp.ceil(counts / TM)
    tile_cum = jnp.cumsum(tiles_per)
    tile_first = tile_cum - tiles_per
    pos_a = (onehot @ (tile_first * TM) + rank_a).astype(i32)

    t_ids = jnp.arange(NT, dtype=f32)
    te = jnp.minimum(jnp.sum(
        (tile_cum[None, :] <= t_ids[:, None]).astype(f32), axis=1),
        E - 1)
    te_oh = (te[:, None] == jnp.arange(E, dtype=f32)[None, :]).astype(f32)
    counts_te = te_oh @ counts
    first_te = te_oh @ tile_first
    tile_valid = jnp.clip(counts_te - (t_ids - first_te) * TM, 0, TM)
    tile_e = te.astype(i32)
    tile_valid = tile_valid.astype(i32)

    tok_a = jnp.concatenate([jnp.arange(T, dtype=i32)] * TOPK)

    w_a = top_w.T.reshape(-1)
    wsrc = jnp.broadcast_to(w_a[:, None], (A, 128))

    xs, wr = _sc_dispatch(x, tok_a, pos_a, wsrc, P)
    ys = _grouped_mlp(xs, w1, w3, w2, wr, tile_e, tile_valid, NT)
    out = _sc_combine(ys, pos_a[:T], pos_a[T:])
    return out.reshape(orig_shape)


# speedup vs baseline: 1.0494x; 1.0494x over previous
import functools

import jax
import jax.numpy as jnp
from jax import lax
from jax.experimental import pallas as pl
from jax.experimental.pallas import tpu as pltpu
from jax.experimental.pallas import tpu_sc as plsc

TOPK = 2
TM = 128


def _router_body(x_ref, gw_ref, idx_ref, w_ref):
    x = x_ref[...]
    gw = gw_ref[...]
    logits = lax.dot_general(x, gw, (((1,), (1,)), ((), ())),
                             preferred_element_type=jnp.float32)
    T, E = logits.shape
    col = lax.broadcasted_iota(jnp.int32, (T, E), 1)
    m1 = jnp.max(logits, axis=1, keepdims=True)
    i1 = jnp.min(jnp.where(logits == m1, col, E), axis=1, keepdims=True)
    l2 = jnp.where(col == i1, -jnp.inf, logits)
    m2 = jnp.max(l2, axis=1, keepdims=True)
    i2 = jnp.min(jnp.where(l2 == m2, col, E), axis=1, keepdims=True)
    p1 = 1.0 / (1.0 + jnp.exp(m2 - m1))
    idx_ref[...] = jnp.concatenate([i1, i2], axis=1)
    w_ref[...] = jnp.concatenate([p1, 1.0 - p1], axis=1)


def _router(x, gate_w):
    T = x.shape[0]
    return pl.pallas_call(
        _router_body,
        out_shape=(jax.ShapeDtypeStruct((T, TOPK), jnp.int32),
                   jax.ShapeDtypeStruct((T, TOPK), jnp.float32)),
    )(x, gate_w)


def _sc_dispatch(x, tok, pos, wsrc, P):
    A = tok.shape[0]
    D = x.shape[1]
    L = wsrc.shape[1]
    info = plsc.get_sparse_core_info()
    NC, NS = info.num_cores, info.num_subcores
    NW = NC * NS
    per_w = A // NW
    CH = 32
    n_chunks = per_w // CH
    pos3 = pos.reshape(NW, n_chunks, CH)
    w4 = wsrc.reshape(NW, n_chunks, CH, L)
    mesh = plsc.VectorSubcoreMesh(core_axis_name="c", subcore_axis_name="s")

    @functools.partial(
        pl.kernel, mesh=mesh,
        out_type=(jax.ShapeDtypeStruct((P, D), jnp.float32),
                  jax.ShapeDtypeStruct((P, L), jnp.float32)),
        scratch_types=[pltpu.VMEM((per_w,), jnp.int32),
                       pltpu.VMEM((CH,), jnp.int32),
                       pltpu.VMEM((CH,), jnp.int32),
                       pltpu.VMEM((CH, L), jnp.float32),
                       pltpu.VMEM((2 * CH, D), jnp.float32),
                       pltpu.SemaphoreType.DMA,
                       pltpu.SemaphoreType.DMA,
                       pltpu.SemaphoreType.DMA,
                       pltpu.SemaphoreType.DMA],
    )
    def dispatch_k(x_hbm, tok_hbm, pos_hbm, w_hbm, out_hbm, wr_hbm,
                   tok_v, pos_v0, pos_v1, w_v, rows_v,
                   gsem0, gsem1, ssem0, ssem1):
        wid = lax.axis_index("s") * NC + lax.axis_index("c")
        base = wid * per_w
        pltpu.sync_copy(tok_hbm.at[pl.ds(base, per_w)], tok_v)
        gsems = (gsem0, gsem1)
        ssems = (ssem0, ssem1)
        posv = (pos_v0, pos_v1)

        def gdesc(i):
            buf = rows_v.at[pl.ds((i % 2) * CH, CH)]
            return pltpu.make_async_copy(
                x_hbm.at[tok_v.at[pl.ds(i * CH, CH)]], buf, gsems[i % 2])

        def sdesc(i):
            buf = rows_v.at[pl.ds((i % 2) * CH, CH)]
            return pltpu.make_async_copy(
                buf, out_hbm.at[posv[i % 2]], ssems[i % 2])

        gdesc(0).start()
        for i in range(n_chunks):
            if i + 1 < n_chunks:
                if i >= 1:
                    sdesc(i - 1).wait()
                gdesc(i + 1).start()
            gdesc(i).wait()
            pltpu.sync_copy(pos_hbm.at[wid, i], posv[i % 2])
            sdesc(i).start()
            pltpu.sync_copy(w_hbm.at[wid, i], w_v)
            pltpu.sync_copy(w_v, wr_hbm.at[posv[i % 2]])
        if n_chunks >= 2:
            sdesc(n_chunks - 2).wait()
        sdesc(n_chunks - 1).wait()

    return dispatch_k(x, tok, pos3, w4)


def _mlp_body(tile_e_ref, valid_ref, xs_ref, w1a_ref, w1b_ref, w3a_ref,
              w3b_ref, w2a_ref, w2b_ref, wr_ref, ys_ref):
    t = pl.program_id(0)

    @pl.when(valid_ref[t] > 0)
    def _():
        xs = xs_ref[...]
        cdim = (((1,), (1,)), ((), ()))
        a1 = lax.dot_general(xs, w1a_ref[0], cdim,
                             preferred_element_type=jnp.float32)
        a2 = lax.dot_general(xs, w1b_ref[0], cdim,
                             preferred_element_type=jnp.float32)
        b1 = lax.dot_general(xs, w3a_ref[0], cdim,
                             preferred_element_type=jnp.float32)
        b2 = lax.dot_general(xs, w3b_ref[0], cdim,
                             preferred_element_type=jnp.float32)
        h1 = (a1 * jax.nn.sigmoid(a1)) * b1
        h2 = (a2 * jax.nn.sigmoid(a2)) * b2
        y = (lax.dot_general(h1, w2a_ref[0], cdim,
                             preferred_element_type=jnp.float32)
             + lax.dot_general(h2, w2b_ref[0], cdim,
                               preferred_element_type=jnp.float32))
        ys_ref[...] = y * wr_ref[:, 0:1]


def _grouped_mlp(xs, w1, w3, w2, wr, tile_e, tile_valid, NT):
    P, d = xs.shape
    E, dff, _ = w1.shape
    dff2 = dff // 2
    grid_spec = pltpu.PrefetchScalarGridSpec(
        num_scalar_prefetch=2,
        grid=(NT,),
        in_specs=[
            pl.BlockSpec((TM, d), lambda i, te, va: (i, 0)),
            pl.BlockSpec((1, dff2, d), lambda i, te, va: (te[i], 0, 0)),
            pl.BlockSpec((1, dff2, d), lambda i, te, va: (te[i], 1, 0)),
            pl.BlockSpec((1, dff2, d), lambda i, te, va: (te[i], 0, 0)),
            pl.BlockSpec((1, dff2, d), lambda i, te, va: (te[i], 1, 0)),
            pl.BlockSpec((1, d, dff2), lambda i, te, va: (te[i], 0, 0)),
            pl.BlockSpec((1, d, dff2), lambda i, te, va: (te[i], 0, 1)),
            pl.BlockSpec((TM, 128), lambda i, te, va: (i, 0)),
        ],
        out_specs=pl.BlockSpec((TM, d), lambda i, te, va: (i, 0)),
    )
    return pl.pallas_call(
        _mlp_body,
        grid_spec=grid_spec,
        out_shape=jax.ShapeDtypeStruct((P, d), jnp.float32),
    )(tile_e, tile_valid, xs, w1, w1, w3, w3, w2, w2, wr)


def _sc_combine(ys, pos0, pos1):
    T = pos0.shape[0]
    D = ys.shape[1]
    info = plsc.get_sparse_core_info()
    NC, NS = info.num_cores, info.num_subcores
    NW = NC * NS
    per_w = T // NW
    CH = 32 if per_w % 32 == 0 else 16
    n_chunks = per_w // CH
    mesh = plsc.VectorSubcoreMesh(core_axis_name="c", subcore_axis_name="s")

    @functools.partial(
        pl.kernel, mesh=mesh,
        out_type=jax.ShapeDtypeStruct((T, D), jnp.float32),
        scratch_types=[pltpu.VMEM((CH,), jnp.int32),
                       pltpu.VMEM((CH,), jnp.int32),
                       pltpu.VMEM((CH, D), jnp.float32),
                       pltpu.VMEM((CH, D), jnp.float32),
                       pltpu.SemaphoreType.DMA],
    )
    def combine_k(ys_hbm, p0_hbm, p1_hbm, out_hbm,
                  p0_v, p1_v, r0_v, r1_v, sem):
        wid = lax.axis_index("s") * NC + lax.axis_index("c")
        base = wid * per_w

        def chunk(c, carry):
            off = base + c * CH
            pltpu.sync_copy(p0_hbm.at[pl.ds(off, CH)], p0_v)
            pltpu.sync_copy(p1_hbm.at[pl.ds(off, CH)], p1_v)
            pltpu.async_copy(ys_hbm.at[p0_v], r0_v, sem).wait()
            pltpu.async_copy(ys_hbm.at[p1_v], r1_v, sem).wait()

            def row(i, rcarry):
                for j in range(D // 16):
                    sl = pl.ds(j * 16, 16)
                    r0_v[i, sl] = r0_v[i, sl] + r1_v[i, sl]
                return rcarry

            lax.fori_loop(0, CH, row, 0)
            pltpu.sync_copy(r0_v, out_hbm.at[pl.ds(off, CH)])
            return carry

        lax.fori_loop(0, n_chunks, chunk, 0)

    return combine_k(ys, pos0, pos1)


def kernel(hidden_states, gate_w, w1, w3, w2):
    orig_shape = hidden_states.shape
    d = orig_shape[-1]
    x = hidden_states.reshape(-1, d)
    T = x.shape[0]
    E = gate_w.shape[0]
    A = T * TOPK
    NT = A // TM + E
    P = NT * TM

    top_idx, top_w = _router(x, gate_w)

    i32 = jnp.int32
    f32 = jnp.float32
    e_a = top_idx.T.reshape(-1)
    onehot = (e_a[:, None] == jnp.arange(E, dtype=i32)[None, :]).astype(f32)
    cum = jnp.cumsum(onehot, axis=0)
    rank_a = jnp.sum((cum - onehot) * onehot, axis=1)
    counts = cum[A - 1]
    tiles_per = jnp.ceil(counts / TM)
    tile_cum = jnp.cumsum(tiles_per)
    tile_first = tile_cum - tiles_per
    pos_a = (onehot @ (tile_first * TM) + rank_a).astype(i32)

    t_ids = jnp.arange(NT, dtype=f32)
    te = jnp.minimum(jnp.sum(
        (tile_cum[None, :] <= t_ids[:, None]).astype(f32), axis=1),
        E - 1)
    te_oh = (te[:, None] == jnp.arange(E, dtype=f32)[None, :]).astype(f32)
    counts_te = te_oh @ counts
    first_te = te_oh @ tile_first
    tile_valid = jnp.clip(counts_te - (t_ids - first_te) * TM, 0, TM)
    tile_e = te.astype(i32)
    tile_valid = tile_valid.astype(i32)

    tok_a = jnp.concatenate([jnp.arange(T, dtype=i32)] * TOPK)

    w_a = top_w.T.reshape(-1)
    wsrc = jnp.broadcast_to(w_a[:, None], (A, 128))

    xs, wr = _sc_dispatch(x, tok_a, pos_a, wsrc, P)
    ys = _grouped_mlp(xs, w1, w3, w2, wr, tile_e, tile_valid, NT)
    out = _sc_combine(ys, pos_a[:T], pos_a[T:])
    return out.reshape(orig_shape)
